# Initial kernel scaffold; baseline (speedup 1.0000x reference)
import jax
import jax.numpy as jnp
from jax import lax
from jax.experimental import pallas as pl
from jax.experimental.pallas import tpu as pltpu
from jax.experimental.pallas import tpu_sc as plsc

_B, _N, _T = 64, 325, 12
_SPACE_D, _DAY_D, _WEEK_D = 64, 32, 32
_DAY_V, _WEEK_V = 288, 7
_OUT_D = 1 + _SPACE_D + _DAY_D + _WEEK_D
_R = _B * _N * _T
_NW = 32
_RPW = _R // _NW
_C = 480
_NCHUNK = -(-_RPW // _C)
_LAST_OFF = _RPW - _C


def _body(x_hbm, space_hbm, day_hbm, week_hbm, out_hbm,
          x_v, space_v, day_v, week_v, out_v):
    wid = lax.axis_index("s") * 2 + lax.axis_index("c")
    wbase = wid * _RPW

    pltpu.sync_copy(space_hbm, space_v)
    pltpu.sync_copy(day_hbm, day_v)
    pltpu.sync_copy(week_hbm, week_v)

    def chunk(ci, carry):
        off = jnp.minimum(ci * _C, _LAST_OFF)
        cbase = wbase + off
        pltpu.sync_copy(x_hbm.at[pl.ds(cbase * 3, _C * 3)], x_v)

        def row(i, carry2):
            o = i * _OUT_D
            xo = i * 3
            d = x_v[xo + 1].astype(jnp.int32)
            w = x_v[xo + 2].astype(jnp.int32)
            n = ((cbase + i) // _T) % _N
            out_v[o] = x_v[xo]
            sb = n * _SPACE_D
            for k in range(4):
                out_v[pl.ds(o + 1 + 16 * k, 16)] = space_v[pl.ds(sb + 16 * k, 16)]
            db = d * _DAY_D
            for k in range(2):
                out_v[pl.ds(o + 65 + 16 * k, 16)] = day_v[pl.ds(db + 16 * k, 16)]
            wb = w * _WEEK_D
            for k in range(2):
                out_v[pl.ds(o + 97 + 16 * k, 16)] = week_v[pl.ds(wb + 16 * k, 16)]
            return carry2

        lax.fori_loop(0, _C, row, 0)
        pltpu.sync_copy(out_v, out_hbm.at[pl.ds(cbase * _OUT_D, _C * _OUT_D)])
        return carry

    lax.fori_loop(0, _NCHUNK, chunk, 0)


def kernel(X, spaceInfo, dayInfo, weekInfo):
    x_flat = X.reshape(_R * 3)
    mesh = plsc.VectorSubcoreMesh(core_axis_name="c", subcore_axis_name="s")
    out = pl.kernel(
        _body,
        mesh=mesh,
        out_type=jax.ShapeDtypeStruct((_R * _OUT_D,), jnp.float32),
        scratch_types=[
            pltpu.VMEM((_C * 3,), jnp.float32),
            pltpu.VMEM((_N * _SPACE_D,), jnp.float32),
            pltpu.VMEM((_DAY_V * _DAY_D,), jnp.float32),
            pltpu.VMEM((_WEEK_V * _WEEK_D,), jnp.float32),
            pltpu.VMEM((_C * _OUT_D,), jnp.float32),
        ],
    )(x_flat, spaceInfo.reshape(-1), dayInfo.reshape(-1), weekInfo.reshape(-1))
    return out.reshape(_B, _N, _T, _OUT_D)


# speedup vs baseline: 1.1485x; 1.1485x over previous
import jax
import jax.numpy as jnp
from jax import lax
from jax.experimental import pallas as pl
from jax.experimental.pallas import tpu as pltpu
from jax.experimental.pallas import tpu_sc as plsc

_B, _N, _T = 64, 325, 12
_SPACE_D, _DAY_D, _WEEK_D = 64, 32, 32
_DAY_V, _WEEK_V = 288, 7
_OUT_D = 1 + _SPACE_D + _DAY_D + _WEEK_D
_R = _B * _N * _T
_NW = 32
_RPW = _R // _NW
_C = 480
_NCHUNK = -(-_RPW // _C)
_LAST_OFF = _RPW - _C
_G = _C // 16


def _body(x_hbm, space_hbm, day_hbm, week_hbm, out_hbm,
          x_v, space_v, day_v, week_v, out_v):
    wid = lax.axis_index("s") * 2 + lax.axis_index("c")
    wbase = wid * _RPW

    pltpu.sync_copy(space_hbm, space_v)
    pltpu.sync_copy(day_hbm, day_v)
    pltpu.sync_copy(week_hbm, week_v)

    lanes = lax.iota(jnp.int32, 16)
    lanes3 = lanes * 3
    lanes_out = lanes * _OUT_D

    def chunk(ci, carry):
        off = jnp.minimum(ci * _C, _LAST_OFF)
        cbase = wbase + off
        pltpu.sync_copy(x_hbm.at[pl.ds(cbase * 3, _C * 3)], x_v)

        def group(g, carry2):
            b = g * 16
            idx0 = b * 3 + lanes3
            fvec = plsc.load_gather(x_v, [idx0])
            dvec = plsc.load_gather(x_v, [idx0 + 1]).astype(jnp.int32)
            wvec = plsc.load_gather(x_v, [idx0 + 2]).astype(jnp.int32)
            plsc.store_scatter(out_v, [b * _OUT_D + lanes_out], fvec)
            for j in range(16):
                o = (b + j) * _OUT_D
                sb = (((cbase + b + j) // _T) % _N) * _SPACE_D
                db = dvec[j] * _DAY_D
                wb = wvec[j] * _WEEK_D
                for k in range(4):
                    out_v[pl.ds(o + 1 + 16 * k, 16)] = space_v[pl.ds(sb + 16 * k, 16)]
                for k in range(2):
                    out_v[pl.ds(o + 65 + 16 * k, 16)] = day_v[pl.ds(db + 16 * k, 16)]
                for k in range(2):
                    out_v[pl.ds(o + 97 + 16 * k, 16)] = week_v[pl.ds(wb + 16 * k, 16)]
            return carry2

        lax.fori_loop(0, _G, group, 0)
        pltpu.sync_copy(out_v, out_hbm.at[pl.ds(cbase * _OUT_D, _C * _OUT_D)])
        return carry

    lax.fori_loop(0, _NCHUNK, chunk, 0)


def kernel(X, spaceInfo, dayInfo, weekInfo):
    x_flat = X.reshape(_R * 3)
    mesh = plsc.VectorSubcoreMesh(core_axis_name="c", subcore_axis_name="s")
    out = pl.kernel(
        _body,
        mesh=mesh,
        compiler_params=pltpu.CompilerParams(needs_layout_passes=False),
        out_type=jax.ShapeDtypeStruct((_R * _OUT_D,), jnp.float32),
        scratch_types=[
            pltpu.VMEM((_C * 3,), jnp.float32),
            pltpu.VMEM((_N * _SPACE_D,), jnp.float32),
            pltpu.VMEM((_DAY_V * _DAY_D,), jnp.float32),
            pltpu.VMEM((_WEEK_V * _WEEK_D,), jnp.float32),
            pltpu.VMEM((_C * _OUT_D,), jnp.float32),
        ],
    )(x_flat, spaceInfo.reshape(-1), dayInfo.reshape(-1), weekInfo.reshape(-1))
    return out.reshape(_B, _N, _T, _OUT_D)
